# Initial kernel scaffold; baseline (speedup 1.0000x reference)
import jax
import jax.numpy as jnp
from jax.experimental import pallas as pl


def kernel(z, weight):
    raise NotImplementedError("write your pallas kernel here")


# speedup vs baseline: 1.0727x; 1.0727x over previous
import functools

import jax
import jax.numpy as jnp
from jax import lax
from jax.experimental import pallas as pl
from jax.experimental.pallas import tpu as pltpu
from jax.experimental.pallas import tpu_sc as plsc

N_EMB = 8192
DIM = 256
BETA = 10.0

M_BLK = 2048
N_BLK = 1024
M_TILES = 16384 // M_BLK
N_TILES = N_EMB // N_BLK

def _argmin_body(z_ref, z2_ref, w_ref, idx_ref, loss_ref, runmin_ref,
                 runarg_ref, runloss_ref):
    m = pl.program_id(0)
    n = pl.program_id(1)

    zb = z_ref[...]
    wb = w_ref[...]

    w2 = jnp.sum(wb * wb, axis=1)
    m2 = lax.dot_general(zb.astype(jnp.bfloat16),
                         (wb + wb).astype(jnp.bfloat16),
                         (((1,), (1,)), ((), ())),
                         preferred_element_type=jnp.float32)
    t1 = z2_ref[...] + w2[None, :]
    d = t1 - m2

    loc = lax.broadcasted_iota(jnp.int32, d.shape, 1)
    ii = loc + n * N_BLK

    @pl.when(n == 0)
    def _():
        runmin_ref[...] = jnp.full((M_BLK, 1), jnp.inf, jnp.float32)
        runarg_ref[...] = jnp.zeros((M_BLK, 1), jnp.int32)
        runloss_ref[...] = jnp.full((M_BLK, 1), jnp.inf, jnp.float32)

    def merge(dv, iv):
        tmin = jnp.min(dv, axis=1, keepdims=True)
        runloss_ref[...] = jnp.minimum(runloss_ref[...], tmin)
        targ = jnp.min(jnp.where(dv == tmin, iv, jnp.int32(2 ** 30)),
                       axis=1, keepdims=True)
        better = tmin < runmin_ref[...]
        runmin_ref[...] = jnp.where(better, tmin, runmin_ref[...])
        runarg_ref[...] = jnp.where(better, targ, runarg_ref[...])

    straddle = jnp.logical_or(n == 2, n == 5)

    @pl.when(jnp.logical_not(straddle))
    def _():
        merge(d, ii)

    @pl.when(straddle)
    def _():
        cut = jnp.where(n == 2, 688, 352)
        in_a = loc < cut
        big = jnp.float32(jnp.inf)
        merge(jnp.where(in_a, d, big), ii)
        runmin_ref[...] = runmin_ref[...].astype(jnp.bfloat16).astype(
            jnp.float32)
        merge(jnp.where(in_a, big, d), ii)

    @pl.when(n == N_TILES - 1)
    def _():
        idx_ref[...] = runarg_ref[...]
        part = jnp.sum(runloss_ref[...])
        prev = jnp.where(m == 0, jnp.float32(0.0), loss_ref[0, 0])
        acc = prev + part
        scale = jnp.float32((1.0 + BETA) / (16384.0 * DIM))
        loss_ref[0, 0] = jnp.where(m == M_TILES - 1, acc * scale, acc)


def _argmin_call(z_flat, z2col, weight):
    return pl.pallas_call(
        _argmin_body,
        grid=(M_TILES, N_TILES),
        in_specs=[
            pl.BlockSpec((M_BLK, DIM), lambda m, n: (m, 0)),
            pl.BlockSpec((M_BLK, 1), lambda m, n: (m, 0)),
            pl.BlockSpec((N_BLK, DIM), lambda m, n: (n, 0)),
        ],
        out_specs=[
            pl.BlockSpec((M_BLK, 1), lambda m, n: (m, 0)),
            pl.BlockSpec(memory_space=pltpu.SMEM, block_shape=(1, 1),
                         index_map=lambda m, n: (0, 0)),
        ],
        out_shape=[
            jax.ShapeDtypeStruct((16384, 1), jnp.int32),
            jax.ShapeDtypeStruct((1, 1), jnp.float32),
        ],
        scratch_shapes=[
            pltpu.VMEM((M_BLK, 1), jnp.float32),
            pltpu.VMEM((M_BLK, 1), jnp.int32),
            pltpu.VMEM((M_BLK, 1), jnp.float32),
        ],
        compiler_params=pltpu.CompilerParams(
            dimension_semantics=("arbitrary", "arbitrary"),
        ),
    )(z_flat, z2col, weight)


_NC = 2
_NS = 16
_NW = _NC * _NS
_B = 16384
_B_PER_W = _B // _NW
_CHUNK = 128
_NCHUNK = _B_PER_W // _CHUNK


def _gather_body(w_hbm, idx_hbm, out_hbm, idx_v, rows_v, sem):
    wid = lax.axis_index("s") * _NC + lax.axis_index("c")
    pltpu.sync_copy(idx_hbm.at[wid], idx_v)
    base = wid * _B_PER_W
    for c in range(_NCHUNK):
        pltpu.async_copy(w_hbm.at[idx_v.at[c]], rows_v, sem).wait()
        pltpu.sync_copy(rows_v, out_hbm.at[pl.ds(base + c * _CHUNK, _CHUNK)])


@functools.partial(jax.jit, static_argnums=())
def _gather_call(weight, idx3):
    mesh = plsc.VectorSubcoreMesh(core_axis_name="c", subcore_axis_name="s")
    k = pl.kernel(
        _gather_body,
        mesh=mesh,
        out_type=jax.ShapeDtypeStruct((_B, DIM), jnp.float32),
        scratch_types=[
            pltpu.VMEM((_NCHUNK, _CHUNK), jnp.int32),
            pltpu.VMEM((_CHUNK, DIM), jnp.float32),
            pltpu.SemaphoreType.DMA,
        ],
    )
    return k(weight, idx3)


def kernel(z, weight):
    z_flat = z.reshape(-1, DIM)
    z2col = jnp.sum(z_flat ** 2, axis=1, keepdims=True)
    idx2d, loss2d = _argmin_call(z_flat, z2col, weight)
    indices_flat = idx2d[:, 0]
    idx3 = indices_flat.reshape(_NW, _NCHUNK, _CHUNK)
    z_q = _gather_call(weight, idx3).reshape(z.shape)
    indices = indices_flat.reshape(z.shape[:-1])
    loss = loss2d[0, 0]
    return (z_q, loss, indices)
